# initial kernel scaffold (unmeasured)
import jax
import jax.numpy as jnp
from jax import lax
from jax.experimental import pallas as pl
from jax.experimental.pallas import tpu as pltpu

N_DEV = 4
B_LOC = 2
SQ = 256
SKV = 256
D_MODEL = 512
HQ_TOT = 16
HQ_LOC = 4
DH = 64
BLK = 64
SCALE = 0.125
NEG = -1e9


def kernel(x, Wq, K_ext, V_ext, Wo):
    my = lax.axis_index("i")
    K_loc = lax.dynamic_slice_in_dim(K_ext, my * B_LOC, B_LOC, axis=0)
    V_loc = lax.dynamic_slice_in_dim(V_ext, my * B_LOC, B_LOC, axis=0)
    K_t = jnp.transpose(K_loc, (2, 0, 1, 3)).reshape(HQ_TOT * B_LOC, SKV, DH)
    V_t = jnp.transpose(V_loc, (2, 0, 1, 3)).reshape(HQ_TOT * B_LOC, SKV, DH)

    def body(x_ref, wq_ref, k_ref, v_ref, wo_ref, out_ref,
             wq_comm, wo_comm, wq_ssem, wq_rsem, wo_ssem, wo_rsem):
        my_pos = lax.axis_index("i")
        right = lax.rem(my_pos + 1, N_DEV)

        qb = lax.broadcasted_iota(jnp.int32, (SQ, SKV), 0) // BLK
        kb = lax.broadcasted_iota(jnp.int32, (SQ, SKV), 1) // BLK
        mask = kb <= qb

        def compute_group(g, wq_g, wo_g, init):
            for b in range(B_LOC):
                xb = x_ref[b]
                q = jnp.dot(xb, wq_g, preferred_element_type=jnp.float32)
                ctx_parts = []
                for hh in range(HQ_LOC):
                    qh = q[:, hh * DH:(hh + 1) * DH]
                    idx = (g * HQ_LOC + hh) * B_LOC + b
                    kh = k_ref[pl.ds(idx, 1)][0]
                    vh = v_ref[pl.ds(idx, 1)][0]
                    s = lax.dot_general(
                        qh, kh, (((1,), (1,)), ((), ())),
                        preferred_element_type=jnp.float32) * SCALE
                    s = jnp.where(mask, s, NEG)
                    m = jnp.max(s, axis=1, keepdims=True)
                    e = jnp.exp(s - m)
                    w = e / jnp.sum(e, axis=1, keepdims=True)
                    ctx_parts.append(
                        jnp.dot(w, vh, preferred_element_type=jnp.float32))
                ctx = jnp.concatenate(ctx_parts, axis=1)
                part = jnp.dot(ctx, wo_g, preferred_element_type=jnp.float32)
                if init:
                    out_ref[b] = part
                else:
                    out_ref[b] = out_ref[b] + part

        wq_comm[0] = wq_ref[...]
        wo_comm[0] = wo_ref[...]

        def make(h, comm, ssem, rsem):
            return pltpu.make_async_remote_copy(
                src_ref=comm.at[h],
                dst_ref=comm.at[h + 1],
                send_sem=ssem.at[h],
                recv_sem=rsem.at[h],
                device_id=(right,),
                device_id_type=pl.DeviceIdType.MESH,
            )

        for h in range(N_DEV - 1):
            r_wq = make(h, wq_comm, wq_ssem, wq_rsem)
            r_wo = make(h, wo_comm, wo_ssem, wo_rsem)
            r_wq.start()
            r_wo.start()
            g = lax.rem(my_pos - h + N_DEV, N_DEV)
            if h == 0:
                compute_group(g, wq_ref[...], wo_ref[...], init=True)
            else:
                compute_group(g, wq_comm[h], wo_comm[h], init=False)
            r_wq.wait()
            r_wo.wait()
        g_last = lax.rem(my_pos - (N_DEV - 1) + N_DEV, N_DEV)
        compute_group(g_last, wq_comm[N_DEV - 1], wo_comm[N_DEV - 1],
                      init=False)

    return pl.pallas_call(
        body,
        out_shape=jax.ShapeDtypeStruct((B_LOC, SQ, D_MODEL), jnp.float32),
        in_specs=[pl.BlockSpec(memory_space=pltpu.VMEM)] * 5,
        out_specs=pl.BlockSpec(memory_space=pltpu.VMEM),
        scratch_shapes=[
            pltpu.VMEM((N_DEV, D_MODEL, HQ_LOC * DH), jnp.float32),
            pltpu.VMEM((N_DEV, HQ_LOC * DH, D_MODEL), jnp.float32),
            pltpu.SemaphoreType.DMA((N_DEV - 1,)),
            pltpu.SemaphoreType.DMA((N_DEV - 1,)),
            pltpu.SemaphoreType.DMA((N_DEV - 1,)),
            pltpu.SemaphoreType.DMA((N_DEV - 1,)),
        ],
        compiler_params=pltpu.CompilerParams(collective_id=0),
    )(x, Wq, K_t, V_t, Wo)


# baseline (device time: 53636 ns/iter reference)
import jax
import jax.numpy as jnp
from jax import lax
from jax.experimental import pallas as pl
from jax.experimental.pallas import tpu as pltpu

N_DEV = 4
B_LOC = 2
SQ = 256
SKV = 256
D_MODEL = 512
HQ_TOT = 16
HQ_LOC = 4
DH = 64
BLK = 64
SCALE = 0.125
NEG = -1e9


def kernel(x, Wq, K_ext, V_ext, Wo):
    my = lax.axis_index("i")
    K_loc = lax.dynamic_slice_in_dim(K_ext, my * B_LOC, B_LOC, axis=0)
    V_loc = lax.dynamic_slice_in_dim(V_ext, my * B_LOC, B_LOC, axis=0)
    K_t = jnp.transpose(K_loc, (2, 0, 1, 3)).reshape(HQ_TOT * B_LOC, SKV, DH)
    V_t = jnp.transpose(V_loc, (2, 0, 1, 3)).reshape(HQ_TOT * B_LOC, SKV, DH)

    def body(x_ref, wq_ref, k_ref, v_ref, wo_ref, out_ref,
             wq_comm, wo_comm, wq_ssem, wq_rsem, wo_ssem, wo_rsem):
        my_pos = lax.axis_index("i")
        right = lax.rem(my_pos + 1, N_DEV)

        qb = lax.broadcasted_iota(jnp.int32, (SQ, SKV), 0) // BLK
        kb = lax.broadcasted_iota(jnp.int32, (SQ, SKV), 1) // BLK
        mask = kb <= qb

        def compute_group(g, wq_g, wo_g, init):
            for b in range(B_LOC):
                xb = x_ref[b]
                q = jnp.dot(xb, wq_g, preferred_element_type=jnp.float32)
                ctx_parts = []
                for hh in range(HQ_LOC):
                    qh = q[:, hh * DH:(hh + 1) * DH]
                    idx = (g * HQ_LOC + hh) * B_LOC + b
                    kh = k_ref[pl.ds(idx, 1)][0]
                    vh = v_ref[pl.ds(idx, 1)][0]
                    s = lax.dot_general(
                        qh, kh, (((1,), (1,)), ((), ())),
                        preferred_element_type=jnp.float32) * SCALE
                    s = jnp.where(mask, s, NEG)
                    m = jnp.max(s, axis=1, keepdims=True)
                    e = jnp.exp(s - m)
                    w = e / jnp.sum(e, axis=1, keepdims=True)
                    ctx_parts.append(
                        jnp.dot(w, vh, preferred_element_type=jnp.float32))
                ctx = jnp.concatenate(ctx_parts, axis=1)
                part = jnp.dot(ctx, wo_g, preferred_element_type=jnp.float32)
                if init:
                    out_ref[b] = part
                else:
                    out_ref[b] = out_ref[b] + part

        wq_comm[0] = wq_ref[...]
        wo_comm[0] = wo_ref[...]

        def make(h, comm, ssem, rsem):
            return pltpu.make_async_remote_copy(
                src_ref=comm.at[h],
                dst_ref=comm.at[h + 1],
                send_sem=ssem.at[h],
                recv_sem=rsem.at[h],
                device_id=(right,),
                device_id_type=pl.DeviceIdType.MESH,
            )

        for h in range(N_DEV - 1):
            r_wq = make(h, wq_comm, wq_ssem, wq_rsem)
            r_wo = make(h, wo_comm, wo_ssem, wo_rsem)
            r_wq.start()
            r_wo.start()
            g = lax.rem(my_pos - h + N_DEV, N_DEV)
            if h == 0:
                compute_group(g, wq_ref[...], wo_ref[...], init=True)
            else:
                compute_group(g, wq_comm[h], wo_comm[h], init=False)
            r_wq.wait()
            r_wo.wait()
        g_last = lax.rem(my_pos - (N_DEV - 1) + N_DEV, N_DEV)
        compute_group(g_last, wq_comm[N_DEV - 1], wo_comm[N_DEV - 1],
                      init=False)

    return pl.pallas_call(
        body,
        out_shape=jax.ShapeDtypeStruct((B_LOC, SQ, D_MODEL), jnp.float32),
        in_specs=[pl.BlockSpec(memory_space=pltpu.VMEM)] * 5,
        out_specs=pl.BlockSpec(memory_space=pltpu.VMEM),
        scratch_shapes=[
            pltpu.VMEM((N_DEV, D_MODEL, HQ_LOC * DH), jnp.float32),
            pltpu.VMEM((N_DEV, HQ_LOC * DH, D_MODEL), jnp.float32),
            pltpu.SemaphoreType.DMA((N_DEV - 1,)),
            pltpu.SemaphoreType.DMA((N_DEV - 1,)),
            pltpu.SemaphoreType.DMA((N_DEV - 1,)),
            pltpu.SemaphoreType.DMA((N_DEV - 1,)),
        ],
    )(x, Wq, K_t, V_t, Wo)


# device time: 24209 ns/iter; 2.2155x vs baseline; 2.2155x over previous
import jax
import jax.numpy as jnp
from jax import lax
from jax.experimental import pallas as pl
from jax.experimental.pallas import tpu as pltpu

N_DEV = 4
B_LOC = 2
SQ = 256
SKV = 256
D_MODEL = 512
HQ_TOT = 16
HQ_LOC = 4
DH = 64
BLK = 64
SCALE = 0.125
NEG = -1e9

COMM_DTYPE = jnp.bfloat16

WQ_HALF = D_MODEL // 2
WO_HALF = (HQ_LOC * DH) // 2


def kernel(x, Wq, K_ext, V_ext, Wo):
    my = lax.axis_index("i")
    K_loc = lax.dynamic_slice_in_dim(K_ext, my * B_LOC, B_LOC, axis=0)
    V_loc = lax.dynamic_slice_in_dim(V_ext, my * B_LOC, B_LOC, axis=0)
    K_t = jnp.transpose(K_loc, (2, 0, 1, 3)).reshape(HQ_TOT * B_LOC, SKV, DH)
    V_t = jnp.transpose(V_loc, (2, 0, 1, 3)).reshape(HQ_TOT * B_LOC, SKV, DH)
    Wq_c = Wq.astype(COMM_DTYPE)
    Wo_c = Wo.astype(COMM_DTYPE)

    def body(x_ref, wq_ref, k_ref, v_ref, wo_ref, out_ref,
             wq_comm, wo_comm, ssem, rsem):
        i = lax.axis_index("i")
        left = lax.rem(i + N_DEV - 1, N_DEV)
        right = lax.rem(i + 1, N_DEV)
        opp = lax.rem(i + 2, N_DEV)

        barrier = pltpu.get_barrier_semaphore()
        for nbr in (left, right):
            pl.semaphore_signal(barrier, inc=1, device_id=(nbr,),
                                device_id_type=pl.DeviceIdType.MESH)
        pl.semaphore_wait(barrier, 2)

        qb = lax.broadcasted_iota(jnp.int32, (SQ, SKV), 0) // BLK
        kb = lax.broadcasted_iota(jnp.int32, (SQ, SKV), 1) // BLK
        mask = kb <= qb

        def compute_group(g, wq_g, wo_g, init):
            wq_g = wq_g.astype(jnp.float32)
            wo_g = wo_g.astype(jnp.float32)
            for b in range(B_LOC):
                xb = x_ref[b]
                q = jnp.dot(xb, wq_g, preferred_element_type=jnp.float32)
                ctx_parts = []
                for hh in range(HQ_LOC):
                    qh = q[:, hh * DH:(hh + 1) * DH]
                    idx = (g * HQ_LOC + hh) * B_LOC + b
                    kh = k_ref[pl.ds(idx, 1)][0]
                    vh = v_ref[pl.ds(idx, 1)][0]
                    s = lax.dot_general(
                        qh, kh, (((1,), (1,)), ((), ())),
                        preferred_element_type=jnp.float32) * SCALE
                    s = jnp.where(mask, s, NEG)
                    m = jnp.max(s, axis=1, keepdims=True)
                    e = jnp.exp(s - m)
                    w = e / jnp.sum(e, axis=1, keepdims=True)
                    ctx_parts.append(
                        jnp.dot(w, vh, preferred_element_type=jnp.float32))
                ctx = jnp.concatenate(ctx_parts, axis=1)
                part = jnp.dot(ctx, wo_g, preferred_element_type=jnp.float32)
                if init:
                    out_ref[b] = part
                else:
                    out_ref[b] = out_ref[b] + part

        def rc(src, dst, si, ri, dev):
            return pltpu.make_async_remote_copy(
                src_ref=src, dst_ref=dst,
                send_sem=ssem.at[si], recv_sem=rsem.at[ri],
                device_id=(dev,), device_id_type=pl.DeviceIdType.MESH)

        cR_wq = rc(wq_ref, wq_comm.at[0], 0, 0, right)
        cR_wo = rc(wo_ref, wo_comm.at[0], 1, 1, right)
        cL_wq = rc(wq_ref, wq_comm.at[1], 2, 2, left)
        cL_wo = rc(wo_ref, wo_comm.at[1], 3, 3, left)
        fR_wq = rc(wq_comm.at[0, pl.ds(0, WQ_HALF)],
                   wq_comm.at[2, pl.ds(0, WQ_HALF)], 4, 4, right)
        fR_wo = rc(wo_comm.at[0, pl.ds(0, WO_HALF)],
                   wo_comm.at[2, pl.ds(0, WO_HALF)], 5, 5, right)
        fL_wq = rc(wq_comm.at[1, pl.ds(WQ_HALF, WQ_HALF)],
                   wq_comm.at[2, pl.ds(WQ_HALF, WQ_HALF)], 6, 6, left)
        fL_wo = rc(wo_comm.at[1, pl.ds(WO_HALF, WO_HALF)],
                   wo_comm.at[2, pl.ds(WO_HALF, WO_HALF)], 7, 7, left)

        cR_wq.start()
        cR_wo.start()
        cL_wq.start()
        cL_wo.start()

        compute_group(i, wq_ref[...], wo_ref[...], init=True)

        cR_wq.wait_recv()
        cR_wo.wait_recv()
        fR_wq.start()
        fR_wo.start()
        cL_wq.wait_recv()
        cL_wo.wait_recv()
        fL_wq.start()
        fL_wo.start()

        compute_group(left, wq_comm[0], wo_comm[0], init=False)
        compute_group(right, wq_comm[1], wo_comm[1], init=False)

        fR_wq.wait_recv()
        fR_wo.wait_recv()
        fL_wq.wait_recv()
        fL_wo.wait_recv()
        compute_group(opp, wq_comm[2], wo_comm[2], init=False)

        for d in (cR_wq, cR_wo, cL_wq, cL_wo, fR_wq, fR_wo, fL_wq, fL_wo):
            d.wait_send()

    return pl.pallas_call(
        body,
        out_shape=jax.ShapeDtypeStruct((B_LOC, SQ, D_MODEL), jnp.float32),
        in_specs=[pl.BlockSpec(memory_space=pltpu.VMEM)] * 5,
        out_specs=pl.BlockSpec(memory_space=pltpu.VMEM),
        scratch_shapes=[
            pltpu.VMEM((3, D_MODEL, HQ_LOC * DH), COMM_DTYPE),
            pltpu.VMEM((3, HQ_LOC * DH, D_MODEL), COMM_DTYPE),
            pltpu.SemaphoreType.DMA((8,)),
            pltpu.SemaphoreType.DMA((8,)),
        ],
        compiler_params=pltpu.CompilerParams(collective_id=0),
    )(x, Wq_c, K_t, V_t, Wo_c)
